# baseline (speedup 1.0000x reference)
import jax
import jax.numpy as jnp
from jax.experimental import pallas as pl
from jax.experimental.pallas import tpu as pltpu

_ROWS = 4096
_CHUNK = 64


def _invmae_body(p_ref, t_ref, out_ref, vacc_ref, cacc_ref):
    i = pl.program_id(0)

    @pl.when(i == 0)
    def _init():
        vacc_ref[...] = jnp.zeros_like(vacc_ref)
        cacc_ref[0] = 0.0

    def body(k, carry):
        aerr, acnt = carry
        sl = pl.ds(k * _CHUNK, _CHUNK)
        p = p_ref[sl, :]
        t = t_ref[sl, :]
        mask = t > 0.0
        err = jnp.where(mask, jnp.abs(p - t) / (p * t), 0.0)
        aerr += jnp.sum(err.reshape(_CHUNK // 8, 8, 512), axis=0)
        acnt += jnp.sum(mask.astype(jnp.float32))
        return aerr, acnt

    zero = jnp.zeros((8, 512), jnp.float32)
    aerr, acnt = jax.lax.fori_loop(
        0, _ROWS // _CHUNK, body, (zero, jnp.float32(0.0))
    )
    vacc_ref[...] += aerr
    cacc_ref[0] += acnt

    @pl.when(i == pl.num_programs(0) - 1)
    def _fin():
        s = jnp.sum(vacc_ref[...])
        c = cacc_ref[0]
        loss = s / jnp.maximum(c, 1.0)
        out_ref[0] = jnp.where(c < 10.0, jnp.float32(-1.0), loss)


def kernel(pred, target):
    n = pred.size
    p = pred.reshape(n // 512, 512)
    t = target.reshape(n // 512, 512)
    grid = n // 512 // _ROWS
    out = pl.pallas_call(
        _invmae_body,
        grid=(grid,),
        in_specs=[
            pl.BlockSpec((_ROWS, 512), lambda i: (i, 0)),
            pl.BlockSpec((_ROWS, 512), lambda i: (i, 0)),
        ],
        out_specs=pl.BlockSpec(memory_space=pltpu.SMEM),
        out_shape=jax.ShapeDtypeStruct((1,), jnp.float32),
        scratch_shapes=[
            pltpu.VMEM((8, 512), jnp.float32),
            pltpu.SMEM((1,), jnp.float32),
        ],
    )(p, t)
    return out[0]


# speedup vs baseline: 2.3214x; 2.3214x over previous
import jax
import jax.numpy as jnp
from jax.experimental import pallas as pl
from jax.experimental.pallas import tpu as pltpu

_ROWS = 4096
_CHUNK = 64


def _invmae_body(p_ref, t_ref, out_ref, vacc_ref, cacc_ref):
    i = pl.program_id(0)

    @pl.when(i == 0)
    def _init():
        vacc_ref[...] = jnp.zeros_like(vacc_ref)
        cacc_ref[...] = jnp.zeros_like(cacc_ref)

    def body(k, carry):
        aerr, acnt = carry
        sl = pl.ds(k * _CHUNK, _CHUNK)
        p = p_ref[sl, :]
        t = t_ref[sl, :]
        mask = t > 0.0
        err = jnp.where(mask, jnp.abs(p - t) / (p * t), 0.0)
        cnt = jnp.where(mask, 1.0, 0.0)
        aerr += jnp.sum(err.reshape(_CHUNK // 8, 8, 512), axis=0)
        acnt += jnp.sum(cnt.reshape(_CHUNK // 8, 8, 512), axis=0)
        return aerr, acnt

    zero = jnp.zeros((8, 512), jnp.float32)
    aerr, acnt = jax.lax.fori_loop(0, _ROWS // _CHUNK, body, (zero, zero))
    vacc_ref[...] += aerr
    cacc_ref[...] += acnt

    @pl.when(i == pl.num_programs(0) - 1)
    def _fin():
        s = jnp.sum(vacc_ref[...])
        c = jnp.sum(cacc_ref[...])
        loss = s / jnp.maximum(c, 1.0)
        out_ref[0] = jnp.where(c < 10.0, jnp.float32(-1.0), loss)


def kernel(pred, target):
    n = pred.size
    p = pred.reshape(n // 512, 512)
    t = target.reshape(n // 512, 512)
    grid = n // 512 // _ROWS
    out = pl.pallas_call(
        _invmae_body,
        grid=(grid,),
        in_specs=[
            pl.BlockSpec((_ROWS, 512), lambda i: (i, 0)),
            pl.BlockSpec((_ROWS, 512), lambda i: (i, 0)),
        ],
        out_specs=pl.BlockSpec(memory_space=pltpu.SMEM),
        out_shape=jax.ShapeDtypeStruct((1,), jnp.float32),
        scratch_shapes=[
            pltpu.VMEM((8, 512), jnp.float32),
            pltpu.VMEM((8, 512), jnp.float32),
        ],
    )(p, t)
    return out[0]


# speedup vs baseline: 2.4173x; 1.0413x over previous
import jax
import jax.numpy as jnp
from jax.experimental import pallas as pl
from jax.experimental.pallas import tpu as pltpu

_ROWS = 4096
_CHUNK = 32


def _invmae_body(p_ref, t_ref, out_ref, vacc_ref, cacc_ref):
    i = pl.program_id(0)

    @pl.when(i == 0)
    def _init():
        vacc_ref[...] = jnp.zeros_like(vacc_ref)
        cacc_ref[...] = jnp.zeros_like(cacc_ref)

    def body(k, carry):
        aerr, acnt = carry
        sl = pl.ds(k * _CHUNK, _CHUNK)
        p = p_ref[sl, :]
        t = t_ref[sl, :]
        mask = t > 0.0
        err = jnp.where(mask, jnp.abs(p - t) / (p * t), 0.0)
        cnt = jnp.where(mask, 1.0, 0.0)
        aerr += jnp.sum(err.reshape(_CHUNK // 8, 8, 512), axis=0)
        acnt += jnp.sum(cnt.reshape(_CHUNK // 8, 8, 512), axis=0)
        return aerr, acnt

    zero = jnp.zeros((8, 512), jnp.float32)
    aerr, acnt = jax.lax.fori_loop(0, _ROWS // _CHUNK, body, (zero, zero))
    vacc_ref[...] += aerr
    cacc_ref[...] += acnt

    @pl.when(i == pl.num_programs(0) - 1)
    def _fin():
        s = jnp.sum(vacc_ref[...])
        c = jnp.sum(cacc_ref[...])
        loss = s / jnp.maximum(c, 1.0)
        out_ref[0] = jnp.where(c < 10.0, jnp.float32(-1.0), loss)


def kernel(pred, target):
    n = pred.size
    p = pred.reshape(n // 512, 512)
    t = target.reshape(n // 512, 512)
    grid = n // 512 // _ROWS
    out = pl.pallas_call(
        _invmae_body,
        grid=(grid,),
        in_specs=[
            pl.BlockSpec((_ROWS, 512), lambda i: (i, 0)),
            pl.BlockSpec((_ROWS, 512), lambda i: (i, 0)),
        ],
        out_specs=pl.BlockSpec(memory_space=pltpu.SMEM),
        out_shape=jax.ShapeDtypeStruct((1,), jnp.float32),
        scratch_shapes=[
            pltpu.VMEM((8, 512), jnp.float32),
            pltpu.VMEM((8, 512), jnp.float32),
        ],
    )(p, t)
    return out[0]


# speedup vs baseline: 2.5273x; 1.0455x over previous
import jax
import jax.numpy as jnp
from jax.experimental import pallas as pl
from jax.experimental.pallas import tpu as pltpu

_ROWS = 1024
_CHUNK = 32


def _invmae_body(p_ref, t_ref, out_ref, vacc_ref, cacc_ref):
    i = pl.program_id(0)

    @pl.when(i == 0)
    def _init():
        vacc_ref[...] = jnp.zeros_like(vacc_ref)
        cacc_ref[...] = jnp.zeros_like(cacc_ref)

    def body(k, carry):
        aerr, acnt = carry
        sl = pl.ds(k * _CHUNK, _CHUNK)
        p = p_ref[sl, :]
        t = t_ref[sl, :]
        mask = t > 0.0
        err = jnp.where(mask, jnp.abs(p - t) / (p * t), 0.0)
        cnt = jnp.where(mask, 1.0, 0.0)
        aerr += jnp.sum(err.reshape(_CHUNK // 8, 8, 512), axis=0)
        acnt += jnp.sum(cnt.reshape(_CHUNK // 8, 8, 512), axis=0)
        return aerr, acnt

    zero = jnp.zeros((8, 512), jnp.float32)
    aerr, acnt = jax.lax.fori_loop(0, _ROWS // _CHUNK, body, (zero, zero))
    vacc_ref[...] += aerr
    cacc_ref[...] += acnt

    @pl.when(i == pl.num_programs(0) - 1)
    def _fin():
        s = jnp.sum(vacc_ref[...])
        c = jnp.sum(cacc_ref[...])
        loss = s / jnp.maximum(c, 1.0)
        out_ref[0] = jnp.where(c < 10.0, jnp.float32(-1.0), loss)


def kernel(pred, target):
    n = pred.size
    p = pred.reshape(n // 512, 512)
    t = target.reshape(n // 512, 512)
    grid = n // 512 // _ROWS
    out = pl.pallas_call(
        _invmae_body,
        grid=(grid,),
        in_specs=[
            pl.BlockSpec((_ROWS, 512), lambda i: (i, 0)),
            pl.BlockSpec((_ROWS, 512), lambda i: (i, 0)),
        ],
        out_specs=pl.BlockSpec(memory_space=pltpu.SMEM),
        out_shape=jax.ShapeDtypeStruct((1,), jnp.float32),
        scratch_shapes=[
            pltpu.VMEM((8, 512), jnp.float32),
            pltpu.VMEM((8, 512), jnp.float32),
        ],
    )(p, t)
    return out[0]
